# baseline (speedup 1.0000x reference)
import functools

import jax
import jax.numpy as jnp
from jax import lax
from jax.experimental import pallas as pl
from jax.experimental.pallas import tpu as pltpu
from jax.experimental.pallas import tpu_sc as plsc

N = 10000
E = 320000
D = 128
N_PAD = 10240
NC, NS = 2, 16
NW = NC * NS
CH = 128
CH_PER_W = 80
E_PAD = NW * CH_PER_W * CH
RPT = N_PAD // NS
CROWS = N_PAD // D
CNT_PAD = 128
BLK = 512


def _pre_body(x_ref, g_ref, b_ref, wl_ref, wr_ref, hl_ref, hr_ref):
    x = x_ref[...]
    mu = jnp.mean(x, axis=-1, keepdims=True)
    xc = x - mu
    var = jnp.mean(xc * xc, axis=-1, keepdims=True)
    h = xc * lax.rsqrt(var + 1e-5) * g_ref[...] + b_ref[...]
    h = jnp.maximum(h, 0.0)
    hl_ref[...] = jnp.dot(h, wl_ref[...], preferred_element_type=jnp.float32)
    hr_ref[...] = jnp.dot(h, wr_ref[...], preferred_element_type=jnp.float32)


def _pre(xp, gamma, beta, wlT, wrT):
    return pl.pallas_call(
        _pre_body,
        grid=(N_PAD // BLK,),
        in_specs=[
            pl.BlockSpec((BLK, D), lambda i: (i, 0)),
            pl.BlockSpec((1, D), lambda i: (0, 0)),
            pl.BlockSpec((1, D), lambda i: (0, 0)),
            pl.BlockSpec((D, D), lambda i: (0, 0)),
            pl.BlockSpec((D, D), lambda i: (0, 0)),
        ],
        out_specs=[
            pl.BlockSpec((BLK, D), lambda i: (i, 0)),
            pl.BlockSpec((BLK, D), lambda i: (i, 0)),
        ],
        out_shape=[
            jax.ShapeDtypeStruct((N_PAD, D), jnp.float32),
            jax.ShapeDtypeStruct((N_PAD, D), jnp.float32),
        ],
    )(xp, gamma.reshape(1, D), beta.reshape(1, D), wlT, wrT)


QCH = 16
CPT0 = 16
CPT1 = 144


def _sc_aggregate(src2d, dst2d, hl, z128, z1d):
    mesh = plsc.VectorSubcoreMesh(core_axis_name="c", subcore_axis_name="s")

    @functools.partial(
        pl.kernel,
        mesh=mesh,
        out_type=[
            jax.ShapeDtypeStruct((N_PAD, D), jnp.float32),
            jax.ShapeDtypeStruct((N_PAD, D), jnp.float32),
            jax.ShapeDtypeStruct((N_PAD,), jnp.float32),
            jax.ShapeDtypeStruct((N_PAD,), jnp.float32),
        ],
        scratch_types=[
            pltpu.VMEM((QCH, 1, CH), jnp.int32),
            pltpu.VMEM((QCH, 1, CH), jnp.int32),
            pltpu.VMEM((CH, D), jnp.float32),
            pltpu.VMEM((CH, D), jnp.float32),
            pltpu.VMEM((CH,), jnp.float32),
            pltpu.VMEM((RPT,), jnp.float32),
            pltpu.VMEM_SHARED((N_PAD, D), jnp.float32),
            pltpu.VMEM_SHARED((N_PAD,), jnp.float32),
            pltpu.SemaphoreType.DMA,
            pltpu.SemaphoreType.DMA,
            pltpu.SemaphoreType.DMA,
        ],
    )
    def k(src_hbm, dst_hbm, hl_hbm, z128_hbm, z1d_hbm,
          acc0_out, acc1_out, cnt0_out, cnt1_out,
          sidx_v, didx_v, r0, r1, ones_v, cbuf_v,
          acc_sh, cnt_sh, g0, g1, sc):
        rows = (r0, r1)
        sems = (g0, g1)
        c = lax.axis_index("c")
        s = lax.axis_index("s")
        base = s * RPT
        nflush = RPT // CH
        nch = lax.select(c == 0, CPT0, CPT1)
        npair = nch // 2
        ebase = lax.select(c == 0, s * CPT0, NS * CPT0 + s * CPT1)

        pltpu.sync_copy(z128_hbm, r0)
        pltpu.sync_copy(z1d_hbm.at[pl.ds(0, RPT)], cbuf_v)
        for j in range(nflush):
            pltpu.sync_copy(r0, acc_sh.at[pl.ds(base + j * CH, CH)])
        pltpu.sync_copy(cbuf_v, cnt_sh.at[pl.ds(base, RPT)])
        for j in range(CH // 16):
            ones_v[pl.ds(j * 16, 16)] = jnp.full((16,), 1.0, dtype=jnp.float32)
        pltpu.sync_copy(src_hbm.at[pl.ds(ebase, QCH)], sidx_v)
        pltpu.sync_copy(dst_hbm.at[pl.ds(ebase, QCH)], didx_v)
        plsc.subcore_barrier()

        pltpu.async_copy(hl_hbm.at[sidx_v.at[0, 0]], r0, g0)
        pltpu.async_copy(hl_hbm.at[sidx_v.at[1, 0]], r1, g1)

        def body(g, carry):
            for b in range(2):
                i = g * 2 + b
                drow = didx_v.at[lax.rem(i, QCH), 0]

                def _drain_cnt():
                    pltpu.make_async_copy(ones_v, cnt_sh.at[drow], sc).wait()
                if b == 0:
                    pl.when(g > 0)(_drain_cnt)
                else:
                    _drain_cnt()

                if b == 0:
                    @pl.when((lax.rem(g, QCH // 2) == 0) & (g > 0))
                    def _():
                        pltpu.sync_copy(
                            dst_hbm.at[pl.ds(ebase + i, QCH)], didx_v)

                pltpu.make_async_copy(
                    hl_hbm.at[pl.ds(0, CH)], rows[b], sems[b]).wait()
                pltpu.sync_copy(rows[b], acc_sh.at[drow], add=True)
                pltpu.async_copy(ones_v, cnt_sh.at[drow], sc, add=True)

                if b == 0:
                    @pl.when((lax.rem(g, QCH // 2) == QCH // 2 - 1)
                             & (g < npair - 1))
                    def _():
                        pltpu.sync_copy(
                            src_hbm.at[pl.ds(ebase + i + 2, QCH)], sidx_v)

                @pl.when(i + 2 < nch)
                def _():
                    pltpu.async_copy(
                        hl_hbm.at[sidx_v.at[lax.rem(i + 2, QCH), 0]],
                        rows[b], sems[b])
            return carry

        lax.fori_loop(0, npair, body, 0)
        pltpu.make_async_copy(ones_v, cnt_sh.at[didx_v.at[0, 0]], sc).wait()
        plsc.subcore_barrier()

        @pl.when(c == 0)
        def _():
            for j in range(nflush):
                off = base + j * CH
                pltpu.sync_copy(acc_sh.at[pl.ds(off, CH)], r0)
                pltpu.sync_copy(r0, acc0_out.at[pl.ds(off, CH)])
            pltpu.sync_copy(cnt_sh.at[pl.ds(base, RPT)], cbuf_v)
            pltpu.sync_copy(cbuf_v, cnt0_out.at[pl.ds(base, RPT)])

        @pl.when(c == 1)
        def _():
            for j in range(nflush):
                off = base + j * CH
                pltpu.sync_copy(acc_sh.at[pl.ds(off, CH)], r0)
                pltpu.sync_copy(r0, acc1_out.at[pl.ds(off, CH)])
            pltpu.sync_copy(cnt_sh.at[pl.ds(base, RPT)], cbuf_v)
            pltpu.sync_copy(cbuf_v, cnt1_out.at[pl.ds(base, RPT)])

    return k(src2d, dst2d, hl, z128, z1d)


def _post_body(a0_ref, a1_ref, c0_ref, c1_ref, hr_ref, bl_ref, o_ref):
    denom = jnp.maximum(c0_ref[...] + c1_ref[...], 1.0)
    o_ref[...] = (a0_ref[...] + a1_ref[...]) / denom + hr_ref[...] + bl_ref[...]


def _post(acc0, acc1, cnt0, cnt1, hr, b_l):
    return pl.pallas_call(
        _post_body,
        grid=(N_PAD // BLK,),
        in_specs=[
            pl.BlockSpec((BLK, D), lambda i: (i, 0)),
            pl.BlockSpec((BLK, D), lambda i: (i, 0)),
            pl.BlockSpec((BLK, 1), lambda i: (i, 0)),
            pl.BlockSpec((BLK, 1), lambda i: (i, 0)),
            pl.BlockSpec((BLK, D), lambda i: (i, 0)),
            pl.BlockSpec((1, D), lambda i: (0, 0)),
        ],
        out_specs=pl.BlockSpec((BLK, D), lambda i: (i, 0)),
        out_shape=jax.ShapeDtypeStruct((N_PAD, D), jnp.float32),
    )(acc0, acc1, cnt0, cnt1, hr, b_l.reshape(1, D))


def kernel(x, edge_index, gamma, beta, W_l, b_l, W_r):
    xp = jnp.pad(x, ((0, N_PAD - N), (0, 0)))
    src = jnp.concatenate([edge_index[0], jnp.zeros((E_PAD - E,), jnp.int32)])
    dst = jnp.concatenate([edge_index[1], jnp.full((E_PAD - E,), N, jnp.int32)])
    src2d = jnp.pad(src.reshape(E_PAD // CH, 1, CH), ((0, QCH), (0, 0), (0, 0)))
    dst2d = jnp.pad(dst.reshape(E_PAD // CH, 1, CH), ((0, QCH), (0, 0), (0, 0)),
                    constant_values=N)
    hl, hr = _pre(xp, gamma, beta, W_l.T, W_r.T)
    z128 = jnp.zeros((CH, D), jnp.float32)
    z1d = jnp.zeros((N_PAD,), jnp.float32)
    acc0, acc1, cnt0, cnt1 = _sc_aggregate(src2d, dst2d, hl, z128, z1d)
    c0 = cnt0.reshape(N_PAD, 1)
    c1 = cnt1.reshape(N_PAD, 1)
    out = _post(acc0, acc1, c0, c1, hr, b_l)
    return out[:N]


# speedup vs baseline: 1.0868x; 1.0868x over previous
import functools

import jax
import jax.numpy as jnp
from jax import lax
from jax.experimental import pallas as pl
from jax.experimental.pallas import tpu as pltpu
from jax.experimental.pallas import tpu_sc as plsc

N = 10000
E = 320000
D = 128
N_PAD = 10240
NC, NS = 2, 16
NW = NC * NS
CH = 128
CH_PER_W = 80
E_PAD = NW * CH_PER_W * CH
RPT = N_PAD // NS
CROWS = N_PAD // D
CNT_PAD = 128
BLK = 512


def _pre_body(x_ref, g_ref, b_ref, wl_ref, wr_ref, hl_ref, hr_ref):
    x = x_ref[...]
    mu = jnp.mean(x, axis=-1, keepdims=True)
    xc = x - mu
    var = jnp.mean(xc * xc, axis=-1, keepdims=True)
    h = xc * lax.rsqrt(var + 1e-5) * g_ref[...] + b_ref[...]
    h = jnp.maximum(h, 0.0)
    hl_ref[...] = jnp.dot(h, wl_ref[...], preferred_element_type=jnp.float32)
    hr_ref[...] = jnp.dot(h, wr_ref[...], preferred_element_type=jnp.float32)


def _pre(xp, gamma, beta, wlT, wrT):
    return pl.pallas_call(
        _pre_body,
        grid=(N_PAD // BLK,),
        in_specs=[
            pl.BlockSpec((BLK, D), lambda i: (i, 0)),
            pl.BlockSpec((1, D), lambda i: (0, 0)),
            pl.BlockSpec((1, D), lambda i: (0, 0)),
            pl.BlockSpec((D, D), lambda i: (0, 0)),
            pl.BlockSpec((D, D), lambda i: (0, 0)),
        ],
        out_specs=[
            pl.BlockSpec((BLK, D), lambda i: (i, 0)),
            pl.BlockSpec((BLK, D), lambda i: (i, 0)),
        ],
        out_shape=[
            jax.ShapeDtypeStruct((N_PAD, D), jnp.float32),
            jax.ShapeDtypeStruct((N_PAD, D), jnp.float32),
        ],
    )(xp, gamma.reshape(1, D), beta.reshape(1, D), wlT, wrT)


QCH = 16
CPT0 = 80
CPT1 = 80


def _sc_aggregate(src2d, dst2d, hl, z128, z1d):
    mesh = plsc.VectorSubcoreMesh(core_axis_name="c", subcore_axis_name="s")

    @functools.partial(
        pl.kernel,
        mesh=mesh,
        out_type=[
            jax.ShapeDtypeStruct((N_PAD, D), jnp.float32),
            jax.ShapeDtypeStruct((N_PAD, D), jnp.float32),
            jax.ShapeDtypeStruct((N_PAD,), jnp.float32),
            jax.ShapeDtypeStruct((N_PAD,), jnp.float32),
        ],
        scratch_types=[
            pltpu.VMEM((QCH, 1, CH), jnp.int32),
            pltpu.VMEM((QCH, 1, CH), jnp.int32),
            pltpu.VMEM((CH, D), jnp.float32),
            pltpu.VMEM((CH, D), jnp.float32),
            pltpu.VMEM((CH,), jnp.float32),
            pltpu.VMEM((RPT,), jnp.float32),
            pltpu.VMEM_SHARED((N_PAD, D), jnp.float32),
            pltpu.VMEM_SHARED((N_PAD,), jnp.float32),
            pltpu.SemaphoreType.DMA,
            pltpu.SemaphoreType.DMA,
            pltpu.SemaphoreType.DMA,
        ],
    )
    def k(src_hbm, dst_hbm, hl_hbm, z128_hbm, z1d_hbm,
          acc0_out, acc1_out, cnt0_out, cnt1_out,
          sidx_v, didx_v, r0, r1, ones_v, cbuf_v,
          acc_sh, cnt_sh, g0, g1, sc):
        rows = (r0, r1)
        sems = (g0, g1)
        c = lax.axis_index("c")
        s = lax.axis_index("s")
        base = s * RPT
        nflush = RPT // CH
        nch = lax.select(c == 0, CPT0, CPT1)
        npair = nch // 2
        ebase = lax.select(c == 0, s * CPT0, NS * CPT0 + s * CPT1)

        pltpu.sync_copy(z128_hbm, r0)
        pltpu.sync_copy(z1d_hbm.at[pl.ds(0, RPT)], cbuf_v)
        for j in range(nflush):
            pltpu.sync_copy(r0, acc_sh.at[pl.ds(base + j * CH, CH)])
        pltpu.sync_copy(cbuf_v, cnt_sh.at[pl.ds(base, RPT)])
        for j in range(CH // 16):
            ones_v[pl.ds(j * 16, 16)] = jnp.full((16,), 1.0, dtype=jnp.float32)
        pltpu.sync_copy(src_hbm.at[pl.ds(ebase, QCH)], sidx_v)
        pltpu.sync_copy(dst_hbm.at[pl.ds(ebase, QCH)], didx_v)
        plsc.subcore_barrier()

        pltpu.async_copy(hl_hbm.at[sidx_v.at[0, 0]], r0, g0)
        pltpu.async_copy(hl_hbm.at[sidx_v.at[1, 0]], r1, g1)

        def body(g, carry):
            for b in range(2):
                i = g * 2 + b
                drow = didx_v.at[lax.rem(i, QCH), 0]

                def _drain_cnt():
                    pltpu.make_async_copy(ones_v, cnt_sh.at[drow], sc).wait()
                if b == 0:
                    pl.when(g > 0)(_drain_cnt)
                else:
                    _drain_cnt()

                if b == 0:
                    @pl.when((lax.rem(g, QCH // 2) == 0) & (g > 0))
                    def _():
                        pltpu.sync_copy(
                            dst_hbm.at[pl.ds(ebase + i, QCH)], didx_v)

                pltpu.make_async_copy(
                    hl_hbm.at[pl.ds(0, CH)], rows[b], sems[b]).wait()
                pltpu.sync_copy(rows[b], acc_sh.at[drow], add=True)
                pltpu.async_copy(ones_v, cnt_sh.at[drow], sc, add=True)

                if b == 0:
                    @pl.when((lax.rem(g, QCH // 2) == QCH // 2 - 1)
                             & (g < npair - 1))
                    def _():
                        pltpu.sync_copy(
                            src_hbm.at[pl.ds(ebase + i + 2, QCH)], sidx_v)

                @pl.when(i + 2 < nch)
                def _():
                    pltpu.async_copy(
                        hl_hbm.at[sidx_v.at[lax.rem(i + 2, QCH), 0]],
                        rows[b], sems[b])
            return carry

        lax.fori_loop(0, npair, body, 0)
        pltpu.make_async_copy(ones_v, cnt_sh.at[didx_v.at[0, 0]], sc).wait()
        plsc.subcore_barrier()

        @pl.when(c == 0)
        def _():
            for j in range(nflush):
                off = base + j * CH
                pltpu.sync_copy(acc_sh.at[pl.ds(off, CH)], r0)
                pltpu.sync_copy(r0, acc0_out.at[pl.ds(off, CH)])
            pltpu.sync_copy(cnt_sh.at[pl.ds(base, RPT)], cbuf_v)
            pltpu.sync_copy(cbuf_v, cnt0_out.at[pl.ds(base, RPT)])

        @pl.when(c == 1)
        def _():
            for j in range(nflush):
                off = base + j * CH
                pltpu.sync_copy(acc_sh.at[pl.ds(off, CH)], r0)
                pltpu.sync_copy(r0, acc1_out.at[pl.ds(off, CH)])
            pltpu.sync_copy(cnt_sh.at[pl.ds(base, RPT)], cbuf_v)
            pltpu.sync_copy(cbuf_v, cnt1_out.at[pl.ds(base, RPT)])

    return k(src2d, dst2d, hl, z128, z1d)


def _post_body(a0_ref, a1_ref, c0_ref, c1_ref, hr_ref, bl_ref, o_ref):
    denom = jnp.maximum(c0_ref[...] + c1_ref[...], 1.0)
    o_ref[...] = (a0_ref[...] + a1_ref[...]) / denom + hr_ref[...] + bl_ref[...]


def _post(acc0, acc1, cnt0, cnt1, hr, b_l):
    return pl.pallas_call(
        _post_body,
        grid=(N_PAD // BLK,),
        in_specs=[
            pl.BlockSpec((BLK, D), lambda i: (i, 0)),
            pl.BlockSpec((BLK, D), lambda i: (i, 0)),
            pl.BlockSpec((BLK, 1), lambda i: (i, 0)),
            pl.BlockSpec((BLK, 1), lambda i: (i, 0)),
            pl.BlockSpec((BLK, D), lambda i: (i, 0)),
            pl.BlockSpec((1, D), lambda i: (0, 0)),
        ],
        out_specs=pl.BlockSpec((BLK, D), lambda i: (i, 0)),
        out_shape=jax.ShapeDtypeStruct((N_PAD, D), jnp.float32),
    )(acc0, acc1, cnt0, cnt1, hr, b_l.reshape(1, D))


def kernel(x, edge_index, gamma, beta, W_l, b_l, W_r):
    xp = jnp.pad(x, ((0, N_PAD - N), (0, 0)))
    src = jnp.concatenate([edge_index[0], jnp.zeros((E_PAD - E,), jnp.int32)])
    dst = jnp.concatenate([edge_index[1], jnp.full((E_PAD - E,), N, jnp.int32)])
    src2d = jnp.pad(src.reshape(E_PAD // CH, 1, CH), ((0, QCH), (0, 0), (0, 0)))
    dst2d = jnp.pad(dst.reshape(E_PAD // CH, 1, CH), ((0, QCH), (0, 0), (0, 0)),
                    constant_values=N)
    hl, hr = _pre(xp, gamma, beta, W_l.T, W_r.T)
    z128 = jnp.zeros((CH, D), jnp.float32)
    z1d = jnp.zeros((N_PAD,), jnp.float32)
    acc0, acc1, cnt0, cnt1 = _sc_aggregate(src2d, dst2d, hl, z128, z1d)
    c0 = cnt0.reshape(N_PAD, 1)
    c1 = cnt1.reshape(N_PAD, 1)
    out = _post(acc0, acc1, c0, c1, hr, b_l)
    return out[:N]
